# baseline (speedup 1.0000x reference)
import functools

import jax
import jax.numpy as jnp
from jax.experimental import pallas as pl
from jax.experimental.pallas import tpu as pltpu

SEQ_LEN = 200
D_MODEL = 128
BATCH = 4096
BB = 128


def _bcast_kernel(pos_ref, idx_ref, out_ref):
    pos = idx_ref[...][:, 0]
    onehot = (pos[:, None] == jax.lax.broadcasted_iota(jnp.int32, (SEQ_LEN, SEQ_LEN), 1)).astype(jnp.float32)
    emb = jax.lax.dot_general(
        onehot, pos_ref[...],
        dimension_numbers=(((1,), (0,)), ((), ())),
        preferred_element_type=jnp.float32,
    )
    out_ref[...] = jnp.broadcast_to(emb[None], (BB, SEQ_LEN, D_MODEL))


@jax.jit
def _run(pos_embed, positions):
    idx2d = positions.astype(jnp.int32).reshape(SEQ_LEN, 1)
    return pl.pallas_call(
        _bcast_kernel,
        grid=(BATCH // BB,),
        in_specs=[
            pl.BlockSpec((SEQ_LEN, D_MODEL), lambda i: (0, 0)),
            pl.BlockSpec((SEQ_LEN, 1), lambda i: (0, 0)),
        ],
        out_specs=pl.BlockSpec((BB, SEQ_LEN, D_MODEL), lambda i: (i, 0, 0)),
        out_shape=jax.ShapeDtypeStruct((BATCH, SEQ_LEN, D_MODEL), jnp.float32),
        compiler_params=pltpu.CompilerParams(
            dimension_semantics=("arbitrary",),
        ),
    )(pos_embed, idx2d)


def kernel(batch_size, pos_embed, positions):
    return _run(pos_embed, positions)
